# baseline (speedup 1.0000x reference)
import functools

import jax
import jax.numpy as jnp
from jax import lax
from jax.experimental import pallas as pl
from jax.experimental.pallas import tpu as pltpu
from jax.experimental.pallas import tpu_sc as plsc

_FEAT = 512
_NBUCKET = 100
_BATCH = 16384
_START_SMOOTH = 1

_NC, _NS, _L = 2, 16, 16
_NW = _NC * _NS
_RPW = _BATCH // _NW
_CH = 8
_NCHUNK = _RPW // _CH
_NBUF = 4


def _tables_body(ep_ref, rv_ref, sv_ref, rm_ref, sm_ref, so_ref):
    live = ep_ref[0, 0] >= _START_SMOOTH
    s = jnp.sqrt(jnp.clip(sv_ref[...] / rv_ref[...], 0.1, 10.0))
    s = jnp.where(live, s, 1.0)
    o = jnp.where(live, sm_ref[...] - rm_ref[...] * s, 0.0)
    so_ref[...] = jnp.concatenate([s, o], axis=1)


def _make_tables(ep, rv, sv, rm, sm):
    return pl.pallas_call(
        _tables_body,
        in_specs=[
            pl.BlockSpec(memory_space=pltpu.SMEM),
            pl.BlockSpec(memory_space=pltpu.VMEM),
            pl.BlockSpec(memory_space=pltpu.VMEM),
            pl.BlockSpec(memory_space=pltpu.VMEM),
            pl.BlockSpec(memory_space=pltpu.VMEM),
        ],
        out_shape=jax.ShapeDtypeStruct((_NBUCKET, 2 * _FEAT), jnp.float32),
    )(ep, rv, sv, rm, sm)


@functools.partial(
    pl.kernel,
    out_type=jax.ShapeDtypeStruct((_BATCH, _FEAT), jnp.float32),
    mesh=plsc.VectorSubcoreMesh(core_axis_name="c", subcore_axis_name="s"),
    scratch_types=[
        pltpu.VMEM((_RPW,), jnp.int32),
        pltpu.VMEM((_NBUF, _CH, _FEAT), jnp.float32),
        pltpu.VMEM((_NBUF, _CH, 2 * _FEAT), jnp.float32),
        pltpu.VMEM((_NBUF, _CH, _FEAT), jnp.float32),
        pltpu.SemaphoreType.DMA,
        pltpu.SemaphoreType.DMA,
        pltpu.SemaphoreType.DMA,
        pltpu.SemaphoreType.DMA,
        pltpu.SemaphoreType.DMA,
        pltpu.SemaphoreType.DMA,
        pltpu.SemaphoreType.DMA,
        pltpu.SemaphoreType.DMA,
    ],
)
def _sc_apply(feat_hbm, lab_hbm, so_hbm, out_hbm,
              idx_v, f_v, t_v, r_v,
              si0, si1, si2, si3, so0, so1, so2, so3):
    sin = (si0, si1, si2, si3)
    sout = (so0, so1, so2, so3)
    wid = lax.axis_index("s") * _NC + lax.axis_index("c")
    base = wid * _RPW
    pltpu.sync_copy(lab_hbm.at[pl.ds(base, _RPW)], idx_v)

    def issue_in(ci, b):
        idx = idx_v.at[pl.ds(ci * _CH, _CH)]
        pltpu.async_copy(so_hbm.at[idx], t_v.at[b], sin[b])
        pltpu.async_copy(feat_hbm.at[pl.ds(base + ci * _CH, _CH)],
                         f_v.at[b], sin[b])

    def wait_in(b):
        pltpu.make_async_copy(so_hbm.at[idx_v.at[pl.ds(0, _CH)]],
                              t_v.at[b], sin[b]).wait()
        pltpu.make_async_copy(feat_hbm.at[pl.ds(base, _CH)],
                              f_v.at[b], sin[b]).wait()

    def wait_out(b):
        pltpu.make_async_copy(r_v.at[b], out_hbm.at[pl.ds(base, _CH)],
                              sout[b]).wait()

    for b in range(_NBUF):
        issue_in(b, b)

    def outer(ci2, carry):
        for b in range(_NBUF):
            ci = ci2 * _NBUF + b
            wait_in(b)

            @pl.when(ci2 > 0)
            def _():
                wait_out(b)

            def row_body(r, c2):
                for j in range(_FEAT // _L):
                    sl = (b, r, pl.ds(j * _L, _L))
                    sc = t_v[b, r, pl.ds(j * _L, _L)]
                    of = t_v[b, r, pl.ds(_FEAT + j * _L, _L)]
                    r_v[sl] = f_v[sl] * sc + of
                return c2

            lax.fori_loop(0, _CH, row_body, 0)
            pltpu.async_copy(r_v.at[b], out_hbm.at[pl.ds(base + ci * _CH, _CH)],
                             sout[b])

            @pl.when(ci + _NBUF < _NCHUNK)
            def _():
                issue_in(ci + _NBUF, b)
        return carry

    lax.fori_loop(0, _NCHUNK // _NBUF, outer, 0)
    for b in range(_NBUF):
        wait_out(b)


def kernel(features, labels, epoch,
           running_mean_last_epoch, running_var_last_epoch,
           smoothed_mean_last_epoch, smoothed_var_last_epoch):
    lab = jnp.clip(labels.reshape(-1).astype(jnp.int32), 0, _NBUCKET - 1)
    ep = jnp.asarray(epoch, jnp.int32).reshape(1, 1)
    so = _make_tables(
        ep, running_var_last_epoch, smoothed_var_last_epoch,
        running_mean_last_epoch, smoothed_mean_last_epoch)
    return _sc_apply(features, lab, so)
